# baseline (speedup 1.0000x reference)
import functools

import jax
import jax.numpy as jnp
from jax import lax
from jax.experimental import pallas as pl
from jax.experimental.pallas import tpu as pltpu
from jax.experimental.pallas import tpu_sc as plsc

_M = 600000
_P1 = 179424941
_P2 = 179425457
_C1 = _P1 % _M
_C2 = _P2 % _M
_C1H = (_C1 * 65536) % _M
_C2H = (_C2 * 65536) % _M

_NC = 2
_NS = 16
_NW = _NC * _NS
_L = 16

_TB = 512
_OFS = 300032
_NBLK = _OFS // _TB


def _relayout(table_t, d):
    def body(a_ref, b_ref, o_ref):
        r = lax.broadcasted_iota(jnp.int32, (d, d), 0)
        c = lax.broadcasted_iota(jnp.int32, (d, d), 1)
        ident = (r == c).astype(jnp.float32)
        dn = (((0,), (0,)), ((), ()))
        o_ref[:, :d] = lax.dot_general(
            a_ref[...], ident, dn, precision=lax.Precision.HIGHEST,
            preferred_element_type=jnp.float32)
        o_ref[:, d:] = lax.dot_general(
            b_ref[...], ident, dn, precision=lax.Precision.HIGHEST,
            preferred_element_type=jnp.float32)

    return pl.pallas_call(
        body,
        grid=(_NBLK,),
        in_specs=[
            pl.BlockSpec((d, _TB), lambda j: (0 * j, j)),
            pl.BlockSpec((d, _TB), lambda j: (0 * j, j + _NBLK)),
        ],
        out_specs=pl.BlockSpec((_TB, 2 * d), lambda j: (j, 0 * j)),
        out_shape=jax.ShapeDtypeStruct((_OFS, 2 * d), jnp.float32),
    )(table_t, table_t)


@functools.partial(jax.jit, static_argnames=("b", "d"))
def _bloom_lookup(indices_i32, table_t, *, b, d):
    table2 = _relayout(table_t, d)
    b_per_w = b // _NW
    n_vec = b_per_w // _L
    mesh = plsc.VectorSubcoreMesh(
        core_axis_name="c", subcore_axis_name="s", num_cores=_NC,
        num_subcores=_NS)

    @functools.partial(
        pl.kernel,
        out_type=jax.ShapeDtypeStruct((b, d), jnp.float32),
        mesh=mesh,
        scratch_types=[
            pltpu.VMEM((b_per_w,), jnp.int32),
            pltpu.VMEM((b_per_w,), jnp.int32),
            pltpu.VMEM((b_per_w,), jnp.int32),
            pltpu.VMEM((b_per_w,), jnp.int32),
            pltpu.VMEM((b_per_w,), jnp.int32),
            pltpu.VMEM((b_per_w // 2, 2 * d), jnp.float32),
            pltpu.VMEM((b_per_w // 2, 2 * d), jnp.float32),
            pltpu.VMEM((b_per_w // 2, d), jnp.float32),
            pltpu.SemaphoreType.DMA,
        ],
        compiler_params=pltpu.CompilerParams(use_tc_tiling_on_sc=True),
    )
    def k(idx_hbm, tab_hbm, out_hbm, idx_v, h1_v, h2_v, o1_v, o2_v,
          r1_v, r2_v, o_v, sem):
        wid = lax.axis_index("s") * jnp.int32(_NC) + lax.axis_index("c")
        base = wid * jnp.int32(b_per_w)
        pltpu.sync_copy(idx_hbm.at[pl.ds(base, b_per_w)], idx_v)

        def hash_body(k_it, _):
            sl = pl.ds(k_it * jnp.int32(_L), _L)
            i = idx_v[sl]
            hi = lax.shift_right_logical(i, jnp.int32(16))
            lo = lax.bitwise_and(i, jnp.int32(0xFFFF))
            m = jnp.int32(_M)
            h1 = (hi * jnp.int32(_C1H) + lo * jnp.int32(_C1)) % m
            h2 = (hi * jnp.int32(_C2H) + lo * jnp.int32(_C2)) % m
            e1 = jnp.where(h1 >= _OFS, jnp.int32(1), jnp.int32(0))
            e2 = jnp.where(h2 >= _OFS, jnp.int32(1), jnp.int32(0))
            h1_v[sl] = h1 - e1 * jnp.int32(_OFS)
            h2_v[sl] = h2 - e2 * jnp.int32(_OFS)
            o1_v[sl] = e1 * jnp.int32(d)
            o2_v[sl] = e2 * jnp.int32(d)
            return _

        lax.fori_loop(jnp.int32(0), jnp.int32(n_vec), hash_body, None)

        half = b_per_w // 2
        for p in range(2):
            pbase = jnp.int32(p * half)

            def issue_body(k_it, _):
                off = k_it * jnp.int32(_L)
                v1 = h1_v[pl.ds(pbase + off, _L)]
                v2 = h2_v[pl.ds(pbase + off, _L)]
                for j in range(_L):
                    pltpu.async_copy(
                        tab_hbm.at[pl.ds(v1[j], 1)],
                        r1_v.at[pl.ds(off + j, 1)], sem)
                    pltpu.async_copy(
                        tab_hbm.at[pl.ds(v2[j], 1)],
                        r2_v.at[pl.ds(off + j, 1)], sem)
                return _

            lax.fori_loop(jnp.int32(0), jnp.int32(half // _L), issue_body,
                          None)
            pltpu.make_async_copy(
                tab_hbm.at[pl.ds(0, half)], r1_v, sem).wait()
            pltpu.make_async_copy(
                tab_hbm.at[pl.ds(0, half)], r2_v, sem).wait()

            def avg_body(k_it, _):
                off = k_it * jnp.int32(_L)
                w1 = o1_v[pl.ds(pbase + off, _L)]
                w2 = o2_v[pl.ds(pbase + off, _L)]
                for j in range(_L):
                    row = off + j
                    o1 = w1[j]
                    o2 = w2[j]
                    for cc in range(d // _L):
                        s = cc * _L
                        o_v[row, pl.ds(s, _L)] = (
                            r1_v[row, pl.ds(o1 + s, _L)] +
                            r2_v[row, pl.ds(o2 + s, _L)]) * 0.5
                return _

            lax.fori_loop(jnp.int32(0), jnp.int32(half // _L), avg_body,
                          None)
            pltpu.sync_copy(o_v, out_hbm.at[pl.ds(base + pbase, half)])

    return k(indices_i32, table2)


def kernel(indices, table):
    b, = indices.shape
    _, d = table.shape
    out = _bloom_lookup(indices.astype(jnp.int32), table.T, b=b, d=d)
    return out.astype(table.dtype)


# speedup vs baseline: 2.3942x; 2.3942x over previous
import functools

import jax
import jax.numpy as jnp
from jax import lax
from jax.experimental import pallas as pl
from jax.experimental.pallas import tpu as pltpu
from jax.experimental.pallas import tpu_sc as plsc

_M = 600000
_P1 = 179424941
_P2 = 179425457
_C1 = _P1 % _M
_C2 = _P2 % _M
_C1H = (_C1 * 65536) % _M
_C2H = (_C2 * 65536) % _M

_NC = 2
_NS = 16
_NW = _NC * _NS
_L = 16


@functools.partial(jax.jit, static_argnames=("b", "d"))
def _bloom_lookup(indices_i32, table, *, b, d):
    b_per_w = b // _NW
    n_vec = b_per_w // _L
    mesh = plsc.VectorSubcoreMesh(
        core_axis_name="c", subcore_axis_name="s", num_cores=_NC,
        num_subcores=_NS)

    @functools.partial(
        pl.kernel,
        out_type=jax.ShapeDtypeStruct((b, d), jnp.float32),
        mesh=mesh,
        scratch_types=[
            pltpu.VMEM((b_per_w,), jnp.int32),
            pltpu.VMEM((b_per_w,), jnp.int32),
            pltpu.VMEM((b_per_w,), jnp.int32),
            pltpu.VMEM((b_per_w // 2, d), jnp.float32),
            pltpu.VMEM((b_per_w // 2, d), jnp.float32),
            pltpu.SemaphoreType.DMA,
        ],
        compiler_params=pltpu.CompilerParams(use_tc_tiling_on_sc=True),
    )
    def k(idx_hbm, table_hbm, out_hbm, idx_v, h1_v, h2_v, r1_v, r2_v, sem):
        wid = lax.axis_index("s") * jnp.int32(_NC) + lax.axis_index("c")
        base = wid * jnp.int32(b_per_w)
        pltpu.sync_copy(idx_hbm.at[pl.ds(base, b_per_w)], idx_v)

        def hash_body(k_it, _):
            sl = pl.ds(k_it * jnp.int32(_L), _L)
            i = idx_v[sl]
            hi = lax.shift_right_logical(i, jnp.int32(16))
            lo = lax.bitwise_and(i, jnp.int32(0xFFFF))
            m = jnp.int32(_M)
            h1_v[sl] = (hi * jnp.int32(_C1H) + lo * jnp.int32(_C1)) % m
            h2_v[sl] = (hi * jnp.int32(_C2H) + lo * jnp.int32(_C2)) % m
            return _

        lax.fori_loop(jnp.int32(0), jnp.int32(n_vec), hash_body, None)

        half = b_per_w // 2
        for p in range(2):
            pbase = p * half

            def issue_body(k_it, _):
                off = k_it * jnp.int32(_L)
                v1 = h1_v[pl.ds(jnp.int32(pbase) + off, _L)]
                v2 = h2_v[pl.ds(jnp.int32(pbase) + off, _L)]
                for j in range(_L):
                    pltpu.async_copy(
                        table_hbm.at[pl.ds(v1[j], 1)],
                        r1_v.at[pl.ds(off + j, 1)], sem)
                    pltpu.async_copy(
                        table_hbm.at[pl.ds(v2[j], 1)],
                        r2_v.at[pl.ds(off + j, 1)], sem)
                return _

            lax.fori_loop(jnp.int32(0), jnp.int32(half // _L), issue_body,
                          None)
            pltpu.make_async_copy(
                table_hbm.at[pl.ds(0, half)], r1_v, sem).wait()
            pltpu.make_async_copy(
                table_hbm.at[pl.ds(0, half)], r2_v, sem).wait()

            def avg_body(row, _):
                for cc in range(d // _L):
                    sl = pl.ds(cc * _L, _L)
                    r1_v[row, sl] = (r1_v[row, sl] + r2_v[row, sl]) * 0.5
                return _

            lax.fori_loop(jnp.int32(0), jnp.int32(half), avg_body, None)
            pltpu.sync_copy(
                r1_v, out_hbm.at[pl.ds(base + jnp.int32(pbase), half)])

    return k(indices_i32, table)


def kernel(indices, table):
    b, = indices.shape
    _, d = table.shape
    out = _bloom_lookup(indices.astype(jnp.int32), table, b=b, d=d)
    return out.astype(table.dtype)
